# baseline (speedup 1.0000x reference)
import functools

import jax
import jax.numpy as jnp
from jax import lax
from jax.experimental import pallas as pl
from jax.experimental.pallas import tpu as pltpu
from jax.experimental.pallas import tpu_sc as plsc

_VOCAB = 1000000
_EMBED = 32
_BATCH = 16384
_LINE = 128
_RPL = _LINE // _EMBED
_NLINES = _VOCAB // _RPL

_NC = 2
_NS = 16
_L = 16
_NW = _NC * _NS
_BPW = _BATCH // _NW
_HALF = _BPW // 2

_mesh = plsc.VectorSubcoreMesh(core_axis_name="c", subcore_axis_name="s")


@functools.partial(
    pl.kernel,
    out_type=jax.ShapeDtypeStruct((_BATCH,), jnp.float32),
    mesh=_mesh,
    scratch_types=[
        pltpu.VMEM((_BPW,), jnp.int32),
        pltpu.VMEM((_BPW,), jnp.int32),
        pltpu.VMEM((_BPW,), jnp.int32),
        pltpu.VMEM((_BPW,), jnp.int32),
        pltpu.VMEM((_HALF, _LINE), jnp.float32),
        pltpu.VMEM((_HALF, _LINE), jnp.float32),
        pltpu.VMEM((_BPW,), jnp.float32),
        pltpu.SemaphoreType.DMA,
        pltpu.SemaphoreType.DMA,
    ],
    compiler_params=pltpu.CompilerParams(
        needs_layout_passes=False, use_tc_tiling_on_sc=True),
)
def _skipgram(qc_hbm, qt_hbm, oc_hbm, ot_hbm, ctx_hbm, tgt_hbm, out_hbm,
              qc_v, qt_v, oc_v, ot_v, ctx_v, tgt_v, out_v, sem_c, sem_t):
    wid = lax.axis_index("s") * _NC + lax.axis_index("c")
    base = wid * _BPW

    pltpu.sync_copy(qc_hbm.at[pl.ds(base, _BPW)], qc_v)
    pltpu.sync_copy(qt_hbm.at[pl.ds(base, _BPW)], qt_v)
    pltpu.sync_copy(oc_hbm.at[pl.ds(base, _BPW)], oc_v)
    pltpu.sync_copy(ot_hbm.at[pl.ds(base, _BPW)], ot_v)

    for h in range(2):
        gc = pltpu.async_copy(
            ctx_hbm.at[qc_v.at[pl.ds(h * _HALF, _HALF)]], ctx_v, sem_c)
        gt = pltpu.async_copy(
            tgt_hbm.at[qt_v.at[pl.ds(h * _HALF, _HALF)]], tgt_v, sem_t)
        gc.wait()
        gt.wait()

        def group(g, carry):
            rows = g * _L + lax.iota(jnp.int32, _L)
            cbase = oc_v[pl.ds(h * _HALF + g * _L, _L)]
            tbase = ot_v[pl.ds(h * _HALF + g * _L, _L)]
            acc = jnp.zeros((_L,), jnp.float32)
            for c in range(_EMBED):
                cv = plsc.load_gather(ctx_v, [rows, cbase + c])
                tv = plsc.load_gather(tgt_v, [rows, tbase + c])
                acc = acc + cv * tv
            out_v[pl.ds(h * _HALF + g * _L, _L)] = 1.0 / (1.0 + jnp.exp(-acc))
            return carry

        lax.fori_loop(0, _HALF // _L, group, 0)

    pltpu.sync_copy(out_v, out_hbm.at[pl.ds(base, _BPW)])


def kernel(x, context_table, target_table):
    xc = x[:, 0].astype(jnp.int32)
    xt = x[:, 1].astype(jnp.int32)
    qc = xc >> 2
    qt = xt >> 2
    oc = (xc & 3) * _EMBED
    ot = (xt & 3) * _EMBED
    ctx = context_table.reshape(_NLINES, _LINE)
    tgt = target_table.reshape(_NLINES, _LINE)
    return _skipgram(qc, qt, oc, ot, ctx, tgt)


# speedup vs baseline: 3.9119x; 3.9119x over previous
import functools

import jax
import jax.numpy as jnp
from jax import lax
from jax.experimental import pallas as pl
from jax.experimental.pallas import tpu as pltpu
from jax.experimental.pallas import tpu_sc as plsc

_VOCAB = 1000000
_EMBED = 32
_BATCH = 16384
_LANE = 128

_NC = 2
_NS = 16
_L = 16
_NW = _NC * _NS
_BPW = _BATCH // _NW
_NG = _BPW // _L
_NSLOT = 8

_mesh = plsc.VectorSubcoreMesh(core_axis_name="c", subcore_axis_name="s")


@functools.partial(
    pl.kernel,
    out_type=jax.ShapeDtypeStruct((_BATCH,), jnp.float32),
    mesh=_mesh,
    scratch_types=[
        pltpu.VMEM((_BPW,), jnp.int32),
        pltpu.VMEM((_BPW,), jnp.int32),
        pltpu.VMEM((_NSLOT * _EMBED, _LANE), jnp.float32),
        pltpu.VMEM((_NSLOT * _EMBED, _LANE), jnp.float32),
        pltpu.VMEM((_BPW,), jnp.float32),
        pltpu.SemaphoreType.DMA((_NSLOT,)),
        pltpu.SemaphoreType.DMA((_NSLOT,)),
    ],
    compiler_params=pltpu.CompilerParams(
        needs_layout_passes=False, use_tc_tiling_on_sc=True),
)
def _skipgram(xc_hbm, xt_hbm, ctxT_hbm, tgtT_hbm, out_hbm,
              xc_v, xt_v, cring, tring, out_v, sem_c, sem_t):
    wid = lax.axis_index("s") * _NC + lax.axis_index("c")
    base = wid * _BPW

    pltpu.sync_copy(xc_hbm.at[pl.ds(base, _BPW)], xc_v)
    pltpu.sync_copy(xt_hbm.at[pl.ds(base, _BPW)], xt_v)

    lane = lax.iota(jnp.int32, _L)

    def issue(slot, ccol, tcol):
        ccol = pl.multiple_of(ccol, _LANE)
        tcol = pl.multiple_of(tcol, _LANE)
        pltpu.async_copy(
            ctxT_hbm.at[:, pl.ds(ccol, _LANE)],
            cring.at[pl.ds(slot * _EMBED, _EMBED)], sem_c.at[slot])
        pltpu.async_copy(
            tgtT_hbm.at[:, pl.ds(tcol, _LANE)],
            tring.at[pl.ds(slot * _EMBED, _EMBED)], sem_t.at[slot])

    def wait(slot):
        pltpu.make_async_copy(
            ctxT_hbm.at[:, pl.ds(0, _LANE)],
            cring.at[pl.ds(slot * _EMBED, _EMBED)], sem_c.at[slot]).wait()
        pltpu.make_async_copy(
            tgtT_hbm.at[:, pl.ds(0, _LANE)],
            tring.at[pl.ds(slot * _EMBED, _EMBED)], sem_t.at[slot]).wait()

    icv0 = xc_v[pl.ds(0, _L)]
    itv0 = xt_v[pl.ds(0, _L)]
    ccol0 = (icv0 >> 7) * _LANE
    tcol0 = (itv0 >> 7) * _LANE
    for j in range(_NSLOT):
        issue(j, ccol0[j], tcol0[j])

    def group(g, carry):
        icv = xc_v[pl.ds(g * _L, _L)]
        itv = xt_v[pl.ds(g * _L, _L)]
        icl = icv & (_LANE - 1)
        itl = itv & (_LANE - 1)
        gn = jnp.minimum(g + 1, _NG - 1)
        icv_n = xc_v[pl.ds(gn * _L, _L)]
        itv_n = xt_v[pl.ds(gn * _L, _L)]
        ccol = (icv >> 7) * _LANE
        tcol = (itv >> 7) * _LANE
        ccol_n = (icv_n >> 7) * _LANE
        tcol_n = (itv_n >> 7) * _LANE

        res = jnp.zeros((_L,), jnp.float32)
        for j in range(_L):
            slot = j % _NSLOT
            wait(slot)
            crows = jnp.full((_L,), slot * _EMBED, jnp.int32) + lane
            trows = crows
            ccols = jnp.full((_L,), 1, jnp.int32) * icl[j]
            tcols = jnp.full((_L,), 1, jnp.int32) * itl[j]
            c0 = plsc.load_gather(cring, [crows, ccols])
            c1 = plsc.load_gather(cring, [crows + _L, ccols])
            t0 = plsc.load_gather(tring, [trows, tcols])
            t1 = plsc.load_gather(tring, [trows + _L, tcols])
            p = c0 * t0 + c1 * t1
            s = lax.reduce_sum_p.bind(p, axes=(0,))
            res = jnp.where(lane == j, s, res)
            if j < _NSLOT:
                issue(slot, ccol[j + _NSLOT], tcol[j + _NSLOT])
            else:
                issue(slot, ccol_n[j - _NSLOT], tcol_n[j - _NSLOT])

        out_v[pl.ds(g * _L, _L)] = 1.0 / (1.0 + jnp.exp(-res))
        return carry

    lax.fori_loop(0, _NG, group, 0)

    for j in range(_NSLOT):
        wait(j)

    pltpu.sync_copy(out_v, out_hbm.at[pl.ds(base, _BPW)])


def kernel(x, context_table, target_table):
    xc = x[:, 0].astype(jnp.int32)
    xt = x[:, 1].astype(jnp.int32)
    return _skipgram(xc, xt, context_table.T, target_table.T)


# speedup vs baseline: 3.9548x; 1.0110x over previous
import functools

import jax
import jax.numpy as jnp
from jax import lax
from jax.experimental import pallas as pl
from jax.experimental.pallas import tpu as pltpu
from jax.experimental.pallas import tpu_sc as plsc

_VOCAB = 1000000
_EMBED = 32
_BATCH = 16384
_LANE = 128

_NC = 2
_NS = 16
_L = 16
_NW = _NC * _NS
_BPW = _BATCH // _NW
_NG = _BPW // _L
_NSLOT = 8

_mesh = plsc.VectorSubcoreMesh(core_axis_name="c", subcore_axis_name="s")


@functools.partial(
    pl.kernel,
    out_type=jax.ShapeDtypeStruct((_BATCH,), jnp.float32),
    mesh=_mesh,
    scratch_types=[
        pltpu.VMEM((_BPW,), jnp.int32),
        pltpu.VMEM((_BPW,), jnp.int32),
        pltpu.VMEM((_NSLOT * _EMBED, _LANE), jnp.float32),
        pltpu.VMEM((_NSLOT * _EMBED, _LANE), jnp.float32),
        pltpu.VMEM((_BPW,), jnp.float32),
        pltpu.SemaphoreType.DMA((_NSLOT,)),
        pltpu.SemaphoreType.DMA((_NSLOT,)),
    ],
    compiler_params=pltpu.CompilerParams(
        needs_layout_passes=False, use_tc_tiling_on_sc=True),
)
def _skipgram(xc_hbm, xt_hbm, ctxT_hbm, tgtT_hbm, out_hbm,
              xc_v, xt_v, cring, tring, out_v, sem_c, sem_t):
    wid = lax.axis_index("s") * _NC + lax.axis_index("c")
    base = wid * _BPW

    pltpu.sync_copy(xc_hbm.at[pl.ds(base, _BPW)], xc_v)
    pltpu.sync_copy(xt_hbm.at[pl.ds(base, _BPW)], xt_v)

    lane = lax.iota(jnp.int32, _L)

    def issue(slot, ccol, tcol):
        ccol = pl.multiple_of(ccol, _LANE)
        tcol = pl.multiple_of(tcol, _LANE)
        for p in range(4):
            pltpu.async_copy(
                ctxT_hbm.at[pl.ds(p * 8, 8), pl.ds(ccol, _LANE)],
                cring.at[pl.ds(slot * _EMBED + p * 8, 8)], sem_c.at[slot])
            pltpu.async_copy(
                tgtT_hbm.at[pl.ds(p * 8, 8), pl.ds(tcol, _LANE)],
                tring.at[pl.ds(slot * _EMBED + p * 8, 8)], sem_t.at[slot])

    def wait(slot):
        pltpu.make_async_copy(
            ctxT_hbm.at[:, pl.ds(0, _LANE)],
            cring.at[pl.ds(slot * _EMBED, _EMBED)], sem_c.at[slot]).wait()
        pltpu.make_async_copy(
            tgtT_hbm.at[:, pl.ds(0, _LANE)],
            tring.at[pl.ds(slot * _EMBED, _EMBED)], sem_t.at[slot]).wait()

    icv0 = xc_v[pl.ds(0, _L)]
    itv0 = xt_v[pl.ds(0, _L)]
    ccol0 = (icv0 >> 7) * _LANE
    tcol0 = (itv0 >> 7) * _LANE
    for j in range(_NSLOT):
        issue(j, ccol0[j], tcol0[j])

    def group(g, carry):
        icv = xc_v[pl.ds(g * _L, _L)]
        itv = xt_v[pl.ds(g * _L, _L)]
        icl = icv & (_LANE - 1)
        itl = itv & (_LANE - 1)
        gn = jnp.minimum(g + 1, _NG - 1)
        icv_n = xc_v[pl.ds(gn * _L, _L)]
        itv_n = xt_v[pl.ds(gn * _L, _L)]
        ccol = (icv >> 7) * _LANE
        tcol = (itv >> 7) * _LANE
        ccol_n = (icv_n >> 7) * _LANE
        tcol_n = (itv_n >> 7) * _LANE

        res = jnp.zeros((_L,), jnp.float32)
        for j in range(_L):
            slot = j % _NSLOT
            wait(slot)
            crows = jnp.full((_L,), slot * _EMBED, jnp.int32) + lane
            trows = crows
            ccols = jnp.full((_L,), 1, jnp.int32) * icl[j]
            tcols = jnp.full((_L,), 1, jnp.int32) * itl[j]
            c0 = plsc.load_gather(cring, [crows, ccols])
            c1 = plsc.load_gather(cring, [crows + _L, ccols])
            t0 = plsc.load_gather(tring, [trows, tcols])
            t1 = plsc.load_gather(tring, [trows + _L, tcols])
            p = c0 * t0 + c1 * t1
            s = lax.reduce_sum_p.bind(p, axes=(0,))
            res = jnp.where(lane == j, s, res)
            if j < _NSLOT:
                issue(slot, ccol[j + _NSLOT], tcol[j + _NSLOT])
            else:
                issue(slot, ccol_n[j - _NSLOT], tcol_n[j - _NSLOT])

        out_v[pl.ds(g * _L, _L)] = 1.0 / (1.0 + jnp.exp(-res))
        return carry

    lax.fori_loop(0, _NG, group, 0)

    for j in range(_NSLOT):
        wait(j)

    pltpu.sync_copy(out_v, out_hbm.at[pl.ds(base, _BPW)])


def kernel(x, context_table, target_table):
    xc = x[:, 0].astype(jnp.int32)
    xt = x[:, 1].astype(jnp.int32)
    return _skipgram(xc, xt, context_table.T, target_table.T)


# speedup vs baseline: 3.9575x; 1.0007x over previous
import functools

import jax
import jax.numpy as jnp
from jax import lax
from jax.experimental import pallas as pl
from jax.experimental.pallas import tpu as pltpu
from jax.experimental.pallas import tpu_sc as plsc

_VOCAB = 1000000
_EMBED = 32
_BATCH = 16384
_LANE = 128

_NC = 2
_NS = 16
_L = 16
_NW = _NC * _NS
_BPW = _BATCH // _NW
_NG = _BPW // _L
_NSLOT = 8

_mesh = plsc.VectorSubcoreMesh(core_axis_name="c", subcore_axis_name="s")


@functools.partial(
    pl.kernel,
    out_type=jax.ShapeDtypeStruct((_BATCH,), jnp.float32),
    mesh=_mesh,
    scratch_types=[
        pltpu.VMEM((_BPW,), jnp.int32),
        pltpu.VMEM((_BPW,), jnp.int32),
        pltpu.VMEM((_NSLOT * _EMBED, _LANE), jnp.float32),
        pltpu.VMEM((_NSLOT * _EMBED, _LANE), jnp.float32),
        pltpu.VMEM((_BPW,), jnp.float32),
        pltpu.SemaphoreType.DMA((_NSLOT,)),
        pltpu.SemaphoreType.DMA((_NSLOT,)),
    ],
    compiler_params=pltpu.CompilerParams(
        needs_layout_passes=False, use_tc_tiling_on_sc=True),
)
def _skipgram(xc_hbm, xt_hbm, ctxT_hbm, tgtT_hbm, out_hbm,
              xc_v, xt_v, cring, tring, out_v, sem_c, sem_t):
    wid = lax.axis_index("s") * _NC + lax.axis_index("c")
    base = wid * _BPW

    pltpu.sync_copy(xc_hbm.at[pl.ds(base, _BPW)], xc_v)
    pltpu.sync_copy(xt_hbm.at[pl.ds(base, _BPW)], xt_v)

    lane = lax.iota(jnp.int32, _L)

    def issue(slot, ccol, tcol):
        ccol = pl.multiple_of(ccol, _LANE)
        tcol = pl.multiple_of(tcol, _LANE)
        for p in range(4):
            pltpu.async_copy(
                ctxT_hbm.at[pl.ds(p * 8, 8), pl.ds(ccol, _LANE)],
                cring.at[pl.ds(slot * _EMBED + p * 8, 8)], sem_c.at[slot])
            pltpu.async_copy(
                tgtT_hbm.at[pl.ds(p * 8, 8), pl.ds(tcol, _LANE)],
                tring.at[pl.ds(slot * _EMBED + p * 8, 8)], sem_t.at[slot])

    def wait(slot):
        pltpu.make_async_copy(
            ctxT_hbm.at[:, pl.ds(0, _LANE)],
            cring.at[pl.ds(slot * _EMBED, _EMBED)], sem_c.at[slot]).wait()
        pltpu.make_async_copy(
            tgtT_hbm.at[:, pl.ds(0, _LANE)],
            tring.at[pl.ds(slot * _EMBED, _EMBED)], sem_t.at[slot]).wait()

    def col_of(iv):
        return (iv >> 7) * _LANE

    icv0 = xc_v[pl.ds(0, _L)]
    itv0 = xt_v[pl.ds(0, _L)]
    ccol0 = col_of(icv0)
    tcol0 = col_of(itv0)
    for j in range(_NSLOT):
        issue(j, ccol0[j], tcol0[j])

    def group(g, carry):
        icv = xc_v[pl.ds(g * _L, _L)]
        itv = xt_v[pl.ds(g * _L, _L)]
        ccol = col_of(icv)
        tcol = col_of(itv)
        icl = icv - ccol
        itl = itv - tcol
        gn = jnp.minimum(g + 1, _NG - 1)
        icv_n = xc_v[pl.ds(gn * _L, _L)]
        itv_n = xt_v[pl.ds(gn * _L, _L)]
        ccol_n = col_of(icv_n)
        tcol_n = col_of(itv_n)

        res = jnp.zeros((_L,), jnp.float32)
        for j in range(_L):
            slot = j % _NSLOT
            wait(slot)
            crows = jnp.full((_L,), slot * _EMBED, jnp.int32) + lane
            trows = crows
            ccols = jnp.full((_L,), 1, jnp.int32) * icl[j]
            tcols = jnp.full((_L,), 1, jnp.int32) * itl[j]
            c0 = plsc.load_gather(cring, [crows, ccols])
            c1 = plsc.load_gather(cring, [crows + _L, ccols])
            t0 = plsc.load_gather(tring, [trows, tcols])
            t1 = plsc.load_gather(tring, [trows + _L, tcols])
            p = c0 * t0 + c1 * t1
            s = lax.reduce_sum_p.bind(p, axes=(0,))
            res = jnp.where(lane == j, s, res)
            if j < _NSLOT:
                issue(slot, ccol[j + _NSLOT], tcol[j + _NSLOT])
            else:
                issue(slot, ccol_n[j - _NSLOT], tcol_n[j - _NSLOT])

        out_v[pl.ds(g * _L, _L)] = 1.0 / (1.0 + jnp.exp(-res))
        return carry

    lax.fori_loop(0, _NG, group, 0)

    for j in range(_NSLOT):
        wait(j)

    pltpu.sync_copy(out_v, out_hbm.at[pl.ds(base, _BPW)])


def kernel(x, context_table, target_table):
    xc = x[:, 0].astype(jnp.int32)
    xt = x[:, 1].astype(jnp.int32)
    return _skipgram(xc, xt, context_table.T, target_table.T)
